# baseline (speedup 1.0000x reference)
import functools

import jax
import jax.numpy as jnp
from jax import lax
from jax.experimental import pallas as pl
from jax.experimental.pallas import tpu as pltpu
from jax.experimental.pallas import tpu_sc as plsc

_E = 8
_D = 1024
_H = 4096
_T = 2048
_K = 2
_TK = _T * _K

_B = 128
_NB = _TK // _B + _E
_CAP = _NB * _B
_HT = 1024
_NH = _H // _HT


def _routing_metadata(expert_idxs, expert_p):
    flat_e = expert_idxs.reshape(-1).astype(jnp.int32)
    oh = (flat_e[:, None] == jnp.arange(_E, dtype=jnp.int32)[None, :])
    cums = jnp.cumsum(oh.astype(jnp.int32), axis=0)
    rank = jnp.take_along_axis(cums, flat_e[:, None], axis=1)[:, 0] - 1
    counts = cums[-1]
    blocks_per_e = (counts + _B - 1) // _B
    block_end = jnp.cumsum(blocks_per_e)
    block_start = block_end - blocks_per_e
    dest = block_start[flat_e] * _B + rank
    src_tok = jnp.arange(_TK, dtype=jnp.int32) // _K
    pad_ids = jnp.arange(_CAP, dtype=jnp.int32) % _T
    src_ids = pad_ids.at[dest].set(src_tok)
    gates = jnp.zeros((_CAP,), jnp.float32).at[dest].set(expert_p.reshape(-1))
    pos = dest
    n_used = block_end[_E - 1]
    b_ids = jnp.arange(_NB, dtype=jnp.int32)
    b_pin = jnp.minimum(b_ids, n_used - 1)
    is_used = (b_ids < n_used).astype(jnp.int32)
    be = jnp.searchsorted(block_end, b_pin, side="right").astype(jnp.int32)
    meta = jnp.stack([be, b_pin, is_used], axis=1)
    return src_ids, gates, pos, meta


_NC = 2
_NS = 16
_NW = _NC * _NS
_GPW = _CAP // _NW
_GCH = _GPW // 2


def _sc_mesh():
    return plsc.VectorSubcoreMesh(core_axis_name="c", subcore_axis_name="s")


_GNC = 4
_GC = _GPW // _GNC


def _gather_rows_body(x_hbm, idx_hbm, out_hbm, idx_v,
                      buf0, buf1, g0, g1, w0, w1):
    wid = lax.axis_index("s") * _NC + lax.axis_index("c")
    base = wid * _GPW
    bufs, gsems, wsems = (buf0, buf1), (g0, g1), (w0, w1)
    pltpu.sync_copy(idx_hbm.at[pl.ds(base, _GPW)], idx_v)

    def gather(c):
        pltpu.async_copy(
            x_hbm.at[idx_v.at[pl.ds(c * _GC, _GC)]],
            bufs[c % 2], gsems[c % 2])

    gather(0)
    for c in range(_GNC):
        pltpu.make_async_copy(
            x_hbm.at[idx_v.at[pl.ds(c * _GC, _GC)]],
            bufs[c % 2], gsems[c % 2]).wait()
        wb = pltpu.async_copy(
            bufs[c % 2], out_hbm.at[pl.ds(base + c * _GC, _GC)],
            wsems[c % 2])
        if c + 1 < _GNC:
            if c >= 1:
                pltpu.make_async_copy(
                    bufs[(c + 1) % 2],
                    out_hbm.at[pl.ds(base + (c - 1) * _GC, _GC)],
                    wsems[(c + 1) % 2]).wait()
            gather(c + 1)
    for c in (_GNC - 2, _GNC - 1):
        pltpu.make_async_copy(
            bufs[c % 2],
            out_hbm.at[pl.ds(base + c * _GC, _GC)],
            wsems[c % 2]).wait()


def _sc_gather_rows(x, src_ids):
    return pl.kernel(
        _gather_rows_body,
        out_type=jax.ShapeDtypeStruct((_CAP, _D), jnp.float32),
        mesh=_sc_mesh(),
        scratch_types=[
            pltpu.VMEM((_GPW,), jnp.int32),
            pltpu.VMEM((_GC, _D), jnp.float32),
            pltpu.VMEM((_GC, _D), jnp.float32),
            pltpu.SemaphoreType.DMA,
            pltpu.SemaphoreType.DMA,
            pltpu.SemaphoreType.DMA,
            pltpu.SemaphoreType.DMA,
        ],
    )(x, src_ids)


_TPW = _T // _NW
_CNC = 4
_TCH = _TPW // _CNC


def _combine_body(ys_hbm, pos_hbm, y_hbm, idx_v,
                  rb0, rb1, ob0, ob1, g0, g1, w0, w1):
    wid = lax.axis_index("s") * _NC + lax.axis_index("c")
    base = wid * _TPW
    rbufs, obufs, gsems, wsems = (rb0, rb1), (ob0, ob1), (g0, g1), (w0, w1)
    pltpu.sync_copy(pos_hbm.at[pl.ds(2 * base, 2 * _TPW)], idx_v)

    def gather(c):
        pltpu.async_copy(
            ys_hbm.at[idx_v.at[pl.ds(c * 2 * _TCH, 2 * _TCH)]],
            rbufs[c % 2], gsems[c % 2])

    gather(0)
    for c in range(_CNC):
        pltpu.make_async_copy(
            ys_hbm.at[idx_v.at[pl.ds(c * 2 * _TCH, 2 * _TCH)]],
            rbufs[c % 2], gsems[c % 2]).wait()
        if c + 1 < _CNC:
            gather(c + 1)
        if c >= 2:
            pltpu.make_async_copy(
                obufs[c % 2],
                y_hbm.at[pl.ds(base + (c - 2) * _TCH, _TCH)],
                wsems[c % 2]).wait()
        rv, ov = rbufs[c % 2], obufs[c % 2]

        def _row(r, _):
            for j in range(_D // 16):
                sl = pl.ds(j * 16, 16)
                ov[r, sl] = rv[2 * r, sl] + rv[2 * r + 1, sl]
            return _

        lax.fori_loop(0, _TCH, _row, 0)
        pltpu.async_copy(
            ov, y_hbm.at[pl.ds(base + c * _TCH, _TCH)], wsems[c % 2])
    for c in (_CNC - 2, _CNC - 1):
        pltpu.make_async_copy(
            obufs[c % 2],
            y_hbm.at[pl.ds(base + c * _TCH, _TCH)],
            wsems[c % 2]).wait()


def _sc_combine(ys, pos):
    return pl.kernel(
        _combine_body,
        out_type=jax.ShapeDtypeStruct((_T, _D), jnp.float32),
        mesh=_sc_mesh(),
        scratch_types=[
            pltpu.VMEM((2 * _TPW,), jnp.int32),
            pltpu.VMEM((2 * _TCH, _D), jnp.float32),
            pltpu.VMEM((2 * _TCH, _D), jnp.float32),
            pltpu.VMEM((_TCH, _D), jnp.float32),
            pltpu.VMEM((_TCH, _D), jnp.float32),
            pltpu.SemaphoreType.DMA,
            pltpu.SemaphoreType.DMA,
            pltpu.SemaphoreType.DMA,
            pltpu.SemaphoreType.DMA,
        ],
    )(ys, pos)


def _gemm_body(meta_ref, xs_ref, g_ref, w1_ref, w2_ref, out_ref, acc_ref):
    h = pl.program_id(0)
    b = pl.program_id(1)

    @pl.when(meta_ref[b, 2] == 1)
    def _():
        xb = xs_ref[...]
        g = g_ref[0, 0, :][:, None]
        ht = jnp.dot(xb, w1_ref[0], preferred_element_type=jnp.float32)
        ht = jnp.maximum(ht, 0.0) * g
        contrib = jnp.dot(ht, w2_ref[0], preferred_element_type=jnp.float32)
        row = meta_ref[b, 1] * _B

        @pl.when(h == 0)
        def _():
            acc_ref[pl.ds(row, _B), :] = contrib

        @pl.when(h != 0)
        def _():
            acc_ref[pl.ds(row, _B), :] += contrib

        @pl.when(h == _NH - 1)
        def _():
            out_ref[...] = acc_ref[pl.ds(row, _B), :]


def _grouped_mlp(xs, gates3, meta, W1, W2, interpret=False):
    grid_spec = pltpu.PrefetchScalarGridSpec(
        num_scalar_prefetch=1,
        grid=(_NH, _NB),
        in_specs=[
            pl.BlockSpec((_B, _D), lambda h, b, m: (m[b, 1], 0)),
            pl.BlockSpec((1, 1, _B), lambda h, b, m: (m[b, 1], 0, 0)),
            pl.BlockSpec((1, _D, _HT), lambda h, b, m: (m[b, 0], 0, h)),
            pl.BlockSpec((1, _HT, _D), lambda h, b, m: (m[b, 0], h, 0)),
        ],
        out_specs=pl.BlockSpec(
            (_B, _D),
            lambda h, b, m: (jnp.where(h == _NH - 1, m[b, 1], _NB), 0)),
        scratch_shapes=[pltpu.VMEM((_CAP, _D), jnp.float32)],
    )
    ys = pl.pallas_call(
        _gemm_body,
        grid_spec=grid_spec,
        out_shape=jax.ShapeDtypeStruct((_CAP + _B, _D), jnp.float32),
        compiler_params=pltpu.CompilerParams(
            dimension_semantics=("arbitrary", "arbitrary"),
            vmem_limit_bytes=60 * 1024 * 1024),
        interpret=interpret,
    )(meta, xs, gates3, W1, W2)
    return ys


def kernel(x, expert_p, expert_idxs, W1, W2):
    src_ids, gates, pos, meta = _routing_metadata(expert_idxs, expert_p)
    xs = _sc_gather_rows(x, src_ids)
    gates3 = gates.reshape(_NB, 1, _B)
    ys = _grouped_mlp(xs, gates3, meta, W1, W2)
    y = _sc_combine(ys, pos)
    return y


# speedup vs baseline: 1.2032x; 1.2032x over previous
import functools

import jax
import jax.numpy as jnp
from jax import lax
from jax.experimental import pallas as pl
from jax.experimental.pallas import tpu as pltpu
from jax.experimental.pallas import tpu_sc as plsc

_E = 8
_D = 1024
_H = 4096
_T = 2048
_K = 2
_TK = _T * _K

_B = 256
_NB = _TK // _B + _E
_CAP = _NB * _B
_HT = 1024
_NH = _H // _HT


def _routing_metadata(expert_idxs, expert_p):
    flat_e = expert_idxs.reshape(-1).astype(jnp.int32)
    oh = (flat_e[:, None] == jnp.arange(_E, dtype=jnp.int32)[None, :])
    cums = jnp.cumsum(oh.astype(jnp.int32), axis=0)
    rank = jnp.take_along_axis(cums, flat_e[:, None], axis=1)[:, 0] - 1
    counts = cums[-1]
    blocks_per_e = (counts + _B - 1) // _B
    block_end = jnp.cumsum(blocks_per_e)
    block_start = block_end - blocks_per_e
    dest = block_start[flat_e] * _B + rank
    src_tok = jnp.arange(_TK, dtype=jnp.int32) // _K
    pad_ids = jnp.arange(_CAP, dtype=jnp.int32) % _T
    src_ids = pad_ids.at[dest].set(src_tok)
    gates = jnp.zeros((_CAP,), jnp.float32).at[dest].set(expert_p.reshape(-1))
    pos = dest
    n_used = block_end[_E - 1]
    b_ids = jnp.arange(_NB, dtype=jnp.int32)
    b_pin = jnp.minimum(b_ids, n_used - 1)
    is_used = (b_ids < n_used).astype(jnp.int32)
    be = jnp.searchsorted(block_end, b_pin, side="right").astype(jnp.int32)
    meta = jnp.stack([be, b_pin, is_used], axis=1)
    return src_ids, gates, pos, meta


_NC = 2
_NS = 16
_NW = _NC * _NS
_GPW = _CAP // _NW
_GCH = _GPW // 2


def _sc_mesh():
    return plsc.VectorSubcoreMesh(core_axis_name="c", subcore_axis_name="s")


_GNC = 4
_GC = _GPW // _GNC


def _gather_rows_body(x_hbm, idx_hbm, out_hbm, idx_v,
                      buf0, buf1, g0, g1, w0, w1):
    wid = lax.axis_index("s") * _NC + lax.axis_index("c")
    base = wid * _GPW
    bufs, gsems, wsems = (buf0, buf1), (g0, g1), (w0, w1)
    pltpu.sync_copy(idx_hbm.at[pl.ds(base, _GPW)], idx_v)

    def gather(c):
        pltpu.async_copy(
            x_hbm.at[idx_v.at[pl.ds(c * _GC, _GC)]],
            bufs[c % 2], gsems[c % 2])

    gather(0)
    for c in range(_GNC):
        pltpu.make_async_copy(
            x_hbm.at[idx_v.at[pl.ds(c * _GC, _GC)]],
            bufs[c % 2], gsems[c % 2]).wait()
        wb = pltpu.async_copy(
            bufs[c % 2], out_hbm.at[pl.ds(base + c * _GC, _GC)],
            wsems[c % 2])
        if c + 1 < _GNC:
            if c >= 1:
                pltpu.make_async_copy(
                    bufs[(c + 1) % 2],
                    out_hbm.at[pl.ds(base + (c - 1) * _GC, _GC)],
                    wsems[(c + 1) % 2]).wait()
            gather(c + 1)
    for c in (_GNC - 2, _GNC - 1):
        pltpu.make_async_copy(
            bufs[c % 2],
            out_hbm.at[pl.ds(base + c * _GC, _GC)],
            wsems[c % 2]).wait()


def _sc_gather_rows(x, src_ids):
    return pl.kernel(
        _gather_rows_body,
        out_type=jax.ShapeDtypeStruct((_CAP, _D), jnp.float32),
        mesh=_sc_mesh(),
        scratch_types=[
            pltpu.VMEM((_GPW,), jnp.int32),
            pltpu.VMEM((_GC, _D), jnp.float32),
            pltpu.VMEM((_GC, _D), jnp.float32),
            pltpu.SemaphoreType.DMA,
            pltpu.SemaphoreType.DMA,
            pltpu.SemaphoreType.DMA,
            pltpu.SemaphoreType.DMA,
        ],
    )(x, src_ids)


_TPW = _T // _NW
_CNC = 4
_TCH = _TPW // _CNC


def _combine_body(ys_hbm, pos_hbm, y_hbm, idx_v,
                  rb0, rb1, ob0, ob1, g0, g1, w0, w1):
    wid = lax.axis_index("s") * _NC + lax.axis_index("c")
    base = wid * _TPW
    rbufs, obufs, gsems, wsems = (rb0, rb1), (ob0, ob1), (g0, g1), (w0, w1)
    pltpu.sync_copy(pos_hbm.at[pl.ds(2 * base, 2 * _TPW)], idx_v)

    def gather(c):
        pltpu.async_copy(
            ys_hbm.at[idx_v.at[pl.ds(c * 2 * _TCH, 2 * _TCH)]],
            rbufs[c % 2], gsems[c % 2])

    gather(0)
    for c in range(_CNC):
        pltpu.make_async_copy(
            ys_hbm.at[idx_v.at[pl.ds(c * 2 * _TCH, 2 * _TCH)]],
            rbufs[c % 2], gsems[c % 2]).wait()
        if c + 1 < _CNC:
            gather(c + 1)
        if c >= 2:
            pltpu.make_async_copy(
                obufs[c % 2],
                y_hbm.at[pl.ds(base + (c - 2) * _TCH, _TCH)],
                wsems[c % 2]).wait()
        rv, ov = rbufs[c % 2], obufs[c % 2]

        def _row(r, _):
            for j in range(_D // 16):
                sl = pl.ds(j * 16, 16)
                ov[r, sl] = rv[2 * r, sl] + rv[2 * r + 1, sl]
            return _

        lax.fori_loop(0, _TCH, _row, 0)
        pltpu.async_copy(
            ov, y_hbm.at[pl.ds(base + c * _TCH, _TCH)], wsems[c % 2])
    for c in (_CNC - 2, _CNC - 1):
        pltpu.make_async_copy(
            obufs[c % 2],
            y_hbm.at[pl.ds(base + c * _TCH, _TCH)],
            wsems[c % 2]).wait()


def _sc_combine(ys, pos):
    return pl.kernel(
        _combine_body,
        out_type=jax.ShapeDtypeStruct((_T, _D), jnp.float32),
        mesh=_sc_mesh(),
        scratch_types=[
            pltpu.VMEM((2 * _TPW,), jnp.int32),
            pltpu.VMEM((2 * _TCH, _D), jnp.float32),
            pltpu.VMEM((2 * _TCH, _D), jnp.float32),
            pltpu.VMEM((_TCH, _D), jnp.float32),
            pltpu.VMEM((_TCH, _D), jnp.float32),
            pltpu.SemaphoreType.DMA,
            pltpu.SemaphoreType.DMA,
            pltpu.SemaphoreType.DMA,
            pltpu.SemaphoreType.DMA,
        ],
    )(ys, pos)


def _gemm_body(meta_ref, xs_ref, g_ref, w1_ref, w2_ref, out_ref, acc_ref):
    h = pl.program_id(0)
    b = pl.program_id(1)

    @pl.when(meta_ref[b, 2] == 1)
    def _():
        xb = xs_ref[...]
        g = g_ref[0, 0, :][:, None]
        ht = jnp.dot(xb, w1_ref[0], preferred_element_type=jnp.float32)
        ht = jnp.maximum(ht, 0.0) * g
        contrib = jnp.dot(ht, w2_ref[0], preferred_element_type=jnp.float32)
        row = meta_ref[b, 1] * _B

        @pl.when(h == 0)
        def _():
            acc_ref[pl.ds(row, _B), :] = contrib

        @pl.when(h != 0)
        def _():
            acc_ref[pl.ds(row, _B), :] += contrib

        @pl.when(h == _NH - 1)
        def _():
            out_ref[...] = acc_ref[pl.ds(row, _B), :]


def _grouped_mlp(xs, gates3, meta, W1, W2, interpret=False):
    grid_spec = pltpu.PrefetchScalarGridSpec(
        num_scalar_prefetch=1,
        grid=(_NH, _NB),
        in_specs=[
            pl.BlockSpec((_B, _D), lambda h, b, m: (m[b, 1], 0)),
            pl.BlockSpec((1, 1, _B), lambda h, b, m: (m[b, 1], 0, 0)),
            pl.BlockSpec((1, _D, _HT), lambda h, b, m: (m[b, 0], 0, h)),
            pl.BlockSpec((1, _HT, _D), lambda h, b, m: (m[b, 0], h, 0)),
        ],
        out_specs=pl.BlockSpec(
            (_B, _D),
            lambda h, b, m: (jnp.where(h == _NH - 1, m[b, 1], _NB), 0)),
        scratch_shapes=[pltpu.VMEM((_CAP, _D), jnp.float32)],
    )
    ys = pl.pallas_call(
        _gemm_body,
        grid_spec=grid_spec,
        out_shape=jax.ShapeDtypeStruct((_CAP + _B, _D), jnp.float32),
        compiler_params=pltpu.CompilerParams(
            dimension_semantics=("arbitrary", "arbitrary"),
            vmem_limit_bytes=60 * 1024 * 1024),
        interpret=interpret,
    )(meta, xs, gates3, W1, W2)
    return ys


def kernel(x, expert_p, expert_idxs, W1, W2):
    src_ids, gates, pos, meta = _routing_metadata(expert_idxs, expert_p)
    xs = _sc_gather_rows(x, src_ids)
    gates3 = gates.reshape(_NB, 1, _B)
    ys = _grouped_mlp(xs, gates3, meta, W1, W2)
    y = _sc_combine(ys, pos)
    return y
